# Initial kernel scaffold; baseline (speedup 1.0000x reference)
import jax
import jax.numpy as jnp
from jax.experimental import pallas as pl


def kernel(x, coeffs, knots):
    raise NotImplementedError("write your pallas kernel here")


# speedup vs baseline: 686.0413x; 686.0413x over previous
import functools

import jax
import jax.numpy as jnp
from jax.experimental import pallas as pl
from jax.experimental.pallas import tpu as pltpu

B = 4096
IN_F = 128
OUT_F = 64
NK = 16
BT = 512


def _kan_body(lo_ref, scale_ref, x_ref, ct_ref, o_ref):
    x = x_ref[...]
    pos = (x - lo_ref[0, 0]) * scale_ref[0, 0]
    id0f = jnp.clip(jnp.floor(pos), 0.0, float(NK - 2))
    t = pos - id0f
    one_m_t = 1.0 - t
    acc = jnp.zeros((x.shape[0], OUT_F), jnp.float32)
    for k in range(NK):
        w = jnp.where(id0f == float(k), one_m_t, 0.0)
        if k >= 1:
            w = w + jnp.where(id0f == float(k - 1), t, 0.0)
        acc = acc + jnp.dot(w, ct_ref[k], preferred_element_type=jnp.float32)
    o_ref[...] = acc


@jax.jit
def kernel(x, coeffs, knots):
    nk = knots.shape[0]
    lo = knots[0].reshape(1, 1)
    scale = ((nk - 1) / (knots[-1] - knots[0])).reshape(1, 1)
    ct = coeffs.transpose(2, 1, 0)
    grid = (B // BT,)
    return pl.pallas_call(
        _kan_body,
        grid=grid,
        in_specs=[
            pl.BlockSpec(memory_space=pltpu.SMEM),
            pl.BlockSpec(memory_space=pltpu.SMEM),
            pl.BlockSpec((BT, IN_F), lambda i: (i, 0)),
            pl.BlockSpec((NK, IN_F, OUT_F), lambda i: (0, 0, 0)),
        ],
        out_specs=pl.BlockSpec((BT, OUT_F), lambda i: (i, 0)),
        out_shape=jax.ShapeDtypeStruct((B, OUT_F), jnp.float32),
    )(lo, scale, x, ct)
